# baseline (speedup 1.0000x reference)
import functools

import jax
import jax.numpy as jnp
from jax import lax
from jax.experimental import pallas as pl
from jax.experimental.pallas import tpu as pltpu
from jax.experimental.pallas import tpu_sc as plsc

_B = 4096
_L = 200
_D = 128
_N_TOTAL = _B * _L
_NC = 2
_NS = 16
_NW = _NC * _NS
_W = _N_TOTAL // _NW
_G = 128
_S = 5
_STEPS = _W // _G
_NOUT = _STEPS // _S


@jax.jit
def _gather(idx_flat, table):
    mesh = plsc.VectorSubcoreMesh(core_axis_name="c", subcore_axis_name="s")

    @functools.partial(
        pl.kernel,
        mesh=mesh,
        out_type=jax.ShapeDtypeStruct((_N_TOTAL, _D), jnp.float32),
        scratch_types=[
            pltpu.VMEM((_W,), jnp.int32),
            pltpu.VMEM((_S * _G, _D), jnp.float32),
            pltpu.SemaphoreType.DMA((_S,)),
            pltpu.SemaphoreType.DMA((_S,)),
        ],
    )
    def k(idx_hbm, tab_hbm, out_hbm, idx_v, rows_v, gsem, osem):
        wid = lax.axis_index("s") * _NC + lax.axis_index("c")
        base = wid * _W

        pltpu.sync_copy(idx_hbm.at[pl.ds(base, _W)], idx_v)

        def outer(m, _):
            t0 = m * _S
            for b in range(_S):
                @pl.when(m > 0)
                def _drain():
                    pltpu.make_async_copy(
                        rows_v.at[pl.ds(b * _G, _G)],
                        out_hbm.at[pl.ds(base + (t0 - _S + b) * _G, _G)],
                        osem.at[b]).wait()
                pltpu.async_copy(
                    tab_hbm.at[idx_v.at[pl.ds((t0 + b) * _G, _G)]],
                    rows_v.at[pl.ds(b * _G, _G)], gsem.at[b])
            for b in range(_S):
                pltpu.make_async_copy(
                    tab_hbm.at[idx_v.at[pl.ds((t0 + b) * _G, _G)]],
                    rows_v.at[pl.ds(b * _G, _G)], gsem.at[b]).wait()
                pltpu.async_copy(
                    rows_v.at[pl.ds(b * _G, _G)],
                    out_hbm.at[pl.ds(base + (t0 + b) * _G, _G)], osem.at[b])
            return 0

        lax.fori_loop(0, _NOUT, outer, 0)

        last_t0 = (_NOUT - 1) * _S
        for b in range(_S):
            pltpu.make_async_copy(
                rows_v.at[pl.ds(b * _G, _G)],
                out_hbm.at[pl.ds(base + (last_t0 + b) * _G, _G)],
                osem.at[b]).wait()

    return k(idx_flat, table)


def kernel(word_idx, emb0, emb1):
    del emb1
    out = _gather(word_idx.reshape(_N_TOTAL), emb0)
    return out.reshape(_B, _L, _D)
